# Initial kernel scaffold; baseline (speedup 1.0000x reference)
import jax
import jax.numpy as jnp
from jax.experimental import pallas as pl


def kernel(emb_s_dynamic, emb_t_dynamic, emb_s_static, emb_t_static, emb_s_icd, emb_t_icd, emb_s_reports, emb_t_reports):
    raise NotImplementedError("write your pallas kernel here")


# speedup vs baseline: 18.3970x; 18.3970x over previous
import functools

import jax
import jax.numpy as jnp
from jax.experimental import pallas as pl
from jax.experimental.pallas import tpu as pltpu

_K = 512
_KEY_NEG_INF = -2139095040
_KEY_POS_INF = 2139095040
_INT32_MIN = -2147483648


def _decode_key(k_int):
    bits = jnp.where(k_int >= 0, k_int, jnp.int32(_INT32_MIN) - k_int)
    return jax.lax.bitcast_convert_type(bits.astype(jnp.int32), jnp.float32)


def _masked_outer(s_row, t_row, n_rows, n_cols, row_offset):
    v = jnp.reshape(s_row, (n_rows, 1)) * jnp.reshape(t_row, (1, n_cols))
    rows = jax.lax.broadcasted_iota(jnp.int32, (n_rows, n_cols), 0) + row_offset
    cols = jax.lax.broadcasted_iota(jnp.int32, (n_rows, n_cols), 1)
    return jnp.where(rows == cols, jnp.float32(-jnp.inf), v)


def _excl_prefix_axis1(x):
    rows, n = x.shape
    incl = x
    d = 1
    while d < n:
        shifted = jnp.concatenate(
            [jnp.zeros((rows, d), x.dtype), incl[:, : n - d]], axis=1
        )
        incl = incl + shifted
        d *= 2
    return incl - x


def _thresh_kernel(s_ref, t_ref, thr_ref, m_ref, r_ref, v_ref):
    n = t_ref.shape[2]
    s = s_ref[0, 0, :]
    t = t_ref[0, 0, :]
    v_ref[...] = _masked_outer(s, t, n, n, 0)

    k_f = jnp.float32(_K)

    def body(_, carry):
        lo, hi = carry
        x = lo ^ hi
        mid = (lo & hi) + (x >> 1) + (x & 1)
        tf = _decode_key(mid)
        cnt = jnp.sum((v_ref[...] >= tf).astype(jnp.float32))
        ge = cnt >= k_f
        return (jnp.where(ge, mid, lo), jnp.where(ge, hi, mid - 1))

    lo, _ = jax.lax.fori_loop(
        0, 33, body, (jnp.int32(_KEY_NEG_INF), jnp.int32(_KEY_POS_INF))
    )
    thr = _decode_key(lo)

    vv = v_ref[...]
    cnt_gt = jnp.sum((vv > thr).astype(jnp.float32))
    eq_rows = jnp.sum((vv == thr).astype(jnp.float32), axis=1, keepdims=True)
    eq_rows = jnp.reshape(eq_rows, (1, n))
    r_ref[0] = _excl_prefix_axis1(eq_rows)

    thr_ref[0] = jnp.full((1, 1), thr, jnp.float32)
    m_ref[0] = jnp.full((1, 1), k_f - cnt_gt, jnp.float32)


def _write_kernel(s_ref, t_ref, thr_ref, m_ref, r_ref, o_ref):
    br = o_ref.shape[1]
    n = o_ref.shape[2]
    b = pl.program_id(1)
    thr = thr_ref[0, 0, 0]
    m = m_ref[0, 0, 0]
    v = _masked_outer(s_ref[0, 0, :], t_ref[0, 0, :], br, n, b * br)
    gt = (v > thr).astype(jnp.float32)
    eq = (v == thr).astype(jnp.float32)
    pref = _excl_prefix_axis1(eq)
    rank = pref + jnp.reshape(r_ref[0, 0, :], (br, 1))
    o_ref[0] = gt + eq * (rank < m).astype(jnp.float32)


@functools.partial(jax.jit, static_argnums=(2, 3))
def _build_adj_mask(emb_s, emb_t, g, n):
    s = emb_s.reshape(g, 1, n)
    t = emb_t.reshape(g, 1, n)

    vec_spec = pl.BlockSpec((1, 1, n), lambda gi: (gi, 0, 0))
    thr, m, r = pl.pallas_call(
        _thresh_kernel,
        grid=(g,),
        in_specs=[vec_spec, vec_spec],
        out_specs=[
            pl.BlockSpec((1, 1, 1), lambda gi: (gi, 0, 0)),
            pl.BlockSpec((1, 1, 1), lambda gi: (gi, 0, 0)),
            pl.BlockSpec((1, 1, n), lambda gi: (gi, 0, 0)),
        ],
        out_shape=[
            jax.ShapeDtypeStruct((g, 1, 1), jnp.float32),
            jax.ShapeDtypeStruct((g, 1, 1), jnp.float32),
            jax.ShapeDtypeStruct((g, 1, n), jnp.float32),
        ],
        scratch_shapes=[pltpu.VMEM((n, n), jnp.float32)],
    )(s, t)

    br = min(n, 256)
    out = pl.pallas_call(
        _write_kernel,
        grid=(g, n // br),
        in_specs=[
            pl.BlockSpec((1, 1, br), lambda gi, bi: (gi, 0, bi)),
            pl.BlockSpec((1, 1, n), lambda gi, bi: (gi, 0, 0)),
            pl.BlockSpec((1, 1, 1), lambda gi, bi: (gi, 0, 0)),
            pl.BlockSpec((1, 1, 1), lambda gi, bi: (gi, 0, 0)),
            pl.BlockSpec((1, 1, br), lambda gi, bi: (gi, 0, bi)),
        ],
        out_specs=pl.BlockSpec((1, br, n), lambda gi, bi: (gi, bi, 0)),
        out_shape=jax.ShapeDtypeStruct((g, n, n), jnp.float32),
    )(s, t, thr, m, r)
    return out


def kernel(emb_s_dynamic, emb_t_dynamic, emb_s_static, emb_t_static,
           emb_s_icd, emb_t_icd, emb_s_reports, emb_t_reports):
    adj_dynamic = _build_adj_mask(emb_s_dynamic, emb_t_dynamic, 8, 1024)
    adj_static = _build_adj_mask(emb_s_static, emb_t_static, 1, 128)
    adj_icd = _build_adj_mask(emb_s_icd, emb_t_icd, 1, 2048)
    adj_reports = _build_adj_mask(emb_s_reports, emb_t_reports, 1, 768)
    return (adj_dynamic, adj_static, adj_icd, adj_reports)


# speedup vs baseline: 30.4463x; 1.6550x over previous
import functools

import jax
import jax.numpy as jnp
from jax.experimental import pallas as pl
from jax.experimental.pallas import tpu as pltpu

_K = 512
_INT32_MIN = -2147483648


def _encode_key(x):
    bits = jax.lax.bitcast_convert_type(x, jnp.int32)
    return jnp.where(bits >= 0, bits, jnp.int32(_INT32_MIN) - bits)


def _decode_key(k_int):
    bits = jnp.where(k_int >= 0, k_int, jnp.int32(_INT32_MIN) - k_int)
    return jax.lax.bitcast_convert_type(bits.astype(jnp.int32), jnp.float32)


def _ceil_avg(lo, hi):
    x = lo ^ hi
    return (lo & hi) + (x >> 1) + (x & 1)


def _masked_outer(s_row, t_row, n_rows, n_cols, row_offset):
    v = jnp.reshape(s_row, (n_rows, 1)) * jnp.reshape(t_row, (1, n_cols))
    rows = jax.lax.broadcasted_iota(jnp.int32, (n_rows, n_cols), 0) + row_offset
    cols = jax.lax.broadcasted_iota(jnp.int32, (n_rows, n_cols), 1)
    return jnp.where(rows == cols, jnp.float32(-jnp.inf), v)


def _excl_prefix_axis1(x):
    rows, n = x.shape
    incl = x
    d = 1
    while d < n:
        shifted = jnp.concatenate(
            [jnp.zeros((rows, d), x.dtype), incl[:, : n - d]], axis=1
        )
        incl = incl + shifted
        d *= 2
    return incl - x


def _thresh_kernel(s_ref, t_ref, thr_ref, m_ref, r_ref, v_ref, sti_ref, stf_ref):
    n = t_ref.shape[2]
    s = s_ref[0, 0, :]
    t = t_ref[0, 0, :]
    v = _masked_outer(s, t, n, n, 0)
    v_ref[...] = v
    k_f = jnp.float32(_K)

    a = jnp.max(jnp.abs(s)) * jnp.max(jnp.abs(t))
    key_neg_a = _encode_key(-a)
    key_pos_a = _encode_key(a)

    if n >= _K:
        rowmax = jnp.reshape(jnp.max(v, axis=1), (1, n))
        colmax = jnp.reshape(jnp.max(v, axis=0), (1, n))

        def sbody(_, c):
            lo1, hi1, lo2, hi2 = c
            mid1 = _ceil_avg(lo1, hi1)
            mid2 = _ceil_avg(lo2, hi2)
            c1 = jnp.sum((rowmax >= _decode_key(mid1)).astype(jnp.float32))
            c2 = jnp.sum((colmax >= _decode_key(mid2)).astype(jnp.float32))
            ge1 = c1 >= k_f
            ge2 = c2 >= k_f
            return (
                jnp.where(ge1, mid1, lo1),
                jnp.where(ge1, hi1, mid1 - 1),
                jnp.where(ge2, mid2, lo2),
                jnp.where(ge2, hi2, mid2 - 1),
            )

        lo1, _, lo2, _ = jax.lax.fori_loop(
            0, 24, sbody, (key_neg_a, key_pos_a, key_neg_a, key_pos_a)
        )
        seed_lo = jnp.maximum(lo1, lo2)
        seed_hi = jnp.maximum(_encode_key(jnp.max(rowmax)), seed_lo)
    else:
        seed_lo = key_neg_a
        seed_hi = key_pos_a

    sti_ref[0] = seed_lo
    sti_ref[1] = seed_hi
    sti_ref[2] = jnp.int32(0)
    stf_ref[0] = jnp.float32(n * n)
    stf_ref[1] = jnp.float32(0.0)
    stf_ref[2] = jnp.float32(0.0)

    def mbody(_, carry):
        @pl.when(sti_ref[2] == 0)
        def _():
            lo = sti_ref[0]
            hi = sti_ref[1]
            cnt_lo = stf_ref[0]
            cnt_hi = stf_ref[1]
            conv = lo >= hi
            hit_lo = jnp.logical_and(jnp.logical_not(conv), cnt_lo == k_f)
            hit_hi = jnp.logical_and(
                jnp.logical_not(jnp.logical_or(conv, hit_lo)),
                (k_f - cnt_hi) == jnp.float32(1.0),
            )
            els = jnp.logical_not(
                jnp.logical_or(conv, jnp.logical_or(hit_lo, hit_hi))
            )

            @pl.when(conv)
            def _():
                stf_ref[2] = _decode_key(lo)
                sti_ref[2] = jnp.int32(1)

            @pl.when(hit_lo)
            def _():
                vlo = _decode_key(lo)
                vv = v_ref[...]
                stf_ref[2] = jnp.min(
                    jnp.where(vv >= vlo, vv, jnp.float32(jnp.inf))
                )
                sti_ref[2] = jnp.int32(1)

            @pl.when(hit_hi)
            def _():
                vhi1 = _decode_key(hi + 1)
                vv = v_ref[...]
                stf_ref[2] = jnp.max(
                    jnp.where(vv < vhi1, vv, jnp.float32(-jnp.inf))
                )
                sti_ref[2] = jnp.int32(1)

            @pl.when(els)
            def _():
                mid = _ceil_avg(lo, hi)
                tf = _decode_key(mid)
                cnt = jnp.sum((v_ref[...] >= tf).astype(jnp.float32))
                ge = cnt >= k_f
                sti_ref[0] = jnp.where(ge, mid, lo)
                sti_ref[1] = jnp.where(ge, hi, mid - 1)
                stf_ref[0] = jnp.where(ge, cnt, cnt_lo)
                stf_ref[1] = jnp.where(ge, cnt_hi, cnt)

        return carry

    jax.lax.fori_loop(0, 40, mbody, jnp.int32(0))

    thr = stf_ref[2]
    vv = v_ref[...]
    cnt_gt = jnp.sum((vv > thr).astype(jnp.float32))
    eq_rows = jnp.sum((vv == thr).astype(jnp.float32), axis=1, keepdims=True)
    eq_rows = jnp.reshape(eq_rows, (1, n))
    r_ref[0] = _excl_prefix_axis1(eq_rows)

    thr_ref[0] = jnp.full((1, 1), thr, jnp.float32)
    m_ref[0] = jnp.full((1, 1), k_f - cnt_gt, jnp.float32)


def _write_kernel(s_ref, t_ref, thr_ref, m_ref, r_ref, o_ref):
    br = o_ref.shape[1]
    n = o_ref.shape[2]
    b = pl.program_id(1)
    thr = thr_ref[0, 0, 0]
    m = m_ref[0, 0, 0]
    v = _masked_outer(s_ref[0, 0, :], t_ref[0, 0, :], br, n, b * br)
    gt = (v > thr).astype(jnp.float32)
    eq = (v == thr).astype(jnp.float32)
    o_ref[0] = gt

    @pl.when(jnp.sum(eq) > 0)
    def _():
        pref = _excl_prefix_axis1(eq)
        rank = pref + jnp.reshape(r_ref[0, 0, :], (br, 1))
        o_ref[0] = gt + eq * (rank < m).astype(jnp.float32)


@functools.partial(jax.jit, static_argnums=(2, 3))
def _build_adj_mask(emb_s, emb_t, g, n):
    s = emb_s.reshape(g, 1, n)
    t = emb_t.reshape(g, 1, n)

    vec_spec = pl.BlockSpec((1, 1, n), lambda gi: (gi, 0, 0))
    thr, m, r = pl.pallas_call(
        _thresh_kernel,
        grid=(g,),
        in_specs=[vec_spec, vec_spec],
        out_specs=[
            pl.BlockSpec((1, 1, 1), lambda gi: (gi, 0, 0)),
            pl.BlockSpec((1, 1, 1), lambda gi: (gi, 0, 0)),
            pl.BlockSpec((1, 1, n), lambda gi: (gi, 0, 0)),
        ],
        out_shape=[
            jax.ShapeDtypeStruct((g, 1, 1), jnp.float32),
            jax.ShapeDtypeStruct((g, 1, 1), jnp.float32),
            jax.ShapeDtypeStruct((g, 1, n), jnp.float32),
        ],
        scratch_shapes=[
            pltpu.VMEM((n, n), jnp.float32),
            pltpu.SMEM((4,), jnp.int32),
            pltpu.SMEM((4,), jnp.float32),
        ],
    )(s, t)

    br = min(n, 256)
    out = pl.pallas_call(
        _write_kernel,
        grid=(g, n // br),
        in_specs=[
            pl.BlockSpec((1, 1, br), lambda gi, bi: (gi, 0, bi)),
            pl.BlockSpec((1, 1, n), lambda gi, bi: (gi, 0, 0)),
            pl.BlockSpec((1, 1, 1), lambda gi, bi: (gi, 0, 0)),
            pl.BlockSpec((1, 1, 1), lambda gi, bi: (gi, 0, 0)),
            pl.BlockSpec((1, 1, br), lambda gi, bi: (gi, 0, bi)),
        ],
        out_specs=pl.BlockSpec((1, br, n), lambda gi, bi: (gi, bi, 0)),
        out_shape=jax.ShapeDtypeStruct((g, n, n), jnp.float32),
    )(s, t, thr, m, r)
    return out


def kernel(emb_s_dynamic, emb_t_dynamic, emb_s_static, emb_t_static,
           emb_s_icd, emb_t_icd, emb_s_reports, emb_t_reports):
    adj_dynamic = _build_adj_mask(emb_s_dynamic, emb_t_dynamic, 8, 1024)
    adj_static = _build_adj_mask(emb_s_static, emb_t_static, 1, 128)
    adj_icd = _build_adj_mask(emb_s_icd, emb_t_icd, 1, 2048)
    adj_reports = _build_adj_mask(emb_s_reports, emb_t_reports, 1, 768)
    return (adj_dynamic, adj_static, adj_icd, adj_reports)
